# baseline (speedup 1.0000x reference)
import jax
import jax.numpy as jnp
from jax import lax
from jax.experimental import pallas as pl
from jax.experimental.pallas import tpu as pltpu
from jax.experimental.pallas import tpu_sc as plsc

D_SINGLE = 384
D_PAIR = 128
NUM_AA = 21
MAX_REL = 32
N_RES = 1024

BI = 4
PAD_ROWS = 2 * N_RES + 2 * MAX_REL


def _pair_kernel(table_ref, out_ref, padded_ref):
    blk = pl.program_id(0)

    @pl.when(blk == 0)
    def _build_padded():
        hi = table_ref[2 * MAX_REL, :]
        lo = table_ref[0, :]
        padded_ref[pl.ds(0, N_RES), :] = jnp.broadcast_to(hi, (N_RES, D_PAIR))
        for r in range(2 * MAX_REL + 1):
            padded_ref[N_RES + r, :] = table_ref[2 * MAX_REL - r, :]
        tail = PAD_ROWS - N_RES - 2 * MAX_REL - 1
        padded_ref[pl.ds(N_RES + 2 * MAX_REL + 1, tail), :] = jnp.broadcast_to(
            lo, (tail, D_PAIR)
        )

    for k in range(BI):
        i = blk * BI + k
        start = (N_RES + MAX_REL) - i
        out_ref[k] = padded_ref[pl.ds(start, N_RES), :]


_N_SC_WORKERS = 32
_ROWS_PER_W = N_RES // _N_SC_WORKERS


def _single_sc_kernel(table_hbm, idx_hbm, out_hbm, idx_v, rows_v, sem):
    wid = lax.axis_index("s") * 2 + lax.axis_index("c")
    base = wid * _ROWS_PER_W
    pltpu.sync_copy(idx_hbm.at[pl.ds(base, _ROWS_PER_W)], idx_v)
    pltpu.async_copy(table_hbm.at[idx_v], rows_v, sem).wait()
    pltpu.sync_copy(rows_v, out_hbm.at[pl.ds(base, _ROWS_PER_W)])


def kernel(target_seq, W_dense, b_dense, relpos_table):
    pair = pl.pallas_call(
        _pair_kernel,
        grid=(N_RES // BI,),
        in_specs=[pl.BlockSpec((2 * MAX_REL + 1, D_PAIR), lambda b: (0, 0))],
        out_specs=pl.BlockSpec((BI, N_RES, D_PAIR), lambda b: (b, 0, 0)),
        out_shape=jax.ShapeDtypeStruct((N_RES, N_RES, D_PAIR), jnp.float32),
        scratch_shapes=[pltpu.VMEM((PAD_ROWS, D_PAIR), jnp.float32)],
    )(relpos_table)

    table_wb = W_dense + b_dense[None, :]
    single = pl.kernel(
        _single_sc_kernel,
        mesh=plsc.VectorSubcoreMesh(core_axis_name="c", subcore_axis_name="s"),
        out_type=jax.ShapeDtypeStruct((N_RES, D_SINGLE), jnp.float32),
        scratch_types=[
            pltpu.VMEM((_ROWS_PER_W,), jnp.int32),
            pltpu.VMEM((_ROWS_PER_W, D_SINGLE), jnp.float32),
            pltpu.SemaphoreType.DMA,
        ],
    )(table_wb, target_seq.astype(jnp.int32))

    return (single, pair)


# speedup vs baseline: 1.1034x; 1.1034x over previous
import jax
import jax.numpy as jnp
from jax import lax
from jax.experimental import pallas as pl
from jax.experimental.pallas import tpu as pltpu
from jax.experimental.pallas import tpu_sc as plsc

D_SINGLE = 384
D_PAIR = 128
NUM_AA = 21
MAX_REL = 32
N_RES = 1024

BI = 8
PAD_ROWS = 2 * N_RES + 2 * MAX_REL


def _pair_kernel(table_ref, out_ref, padded_ref):
    blk = pl.program_id(0)

    @pl.when(blk == 0)
    def _build_padded():
        hi = table_ref[2 * MAX_REL, :]
        lo = table_ref[0, :]
        padded_ref[pl.ds(0, N_RES), :] = jnp.broadcast_to(hi, (N_RES, D_PAIR))
        for r in range(2 * MAX_REL + 1):
            padded_ref[N_RES + r, :] = table_ref[2 * MAX_REL - r, :]
        tail = PAD_ROWS - N_RES - 2 * MAX_REL - 1
        padded_ref[pl.ds(N_RES + 2 * MAX_REL + 1, tail), :] = jnp.broadcast_to(
            lo, (tail, D_PAIR)
        )

    for k in range(BI):
        i = blk * BI + k
        start = (N_RES + MAX_REL) - i
        out_ref[k] = padded_ref[pl.ds(start, N_RES), :]


_N_SC_WORKERS = 32
_ROWS_PER_W = N_RES // _N_SC_WORKERS


def _single_sc_kernel(table_hbm, idx_hbm, bias_hbm, out_hbm, idx_v, rows_v, b_v, sem):
    wid = lax.axis_index("s") * 2 + lax.axis_index("c")
    base = wid * _ROWS_PER_W
    pltpu.sync_copy(idx_hbm.at[pl.ds(base, _ROWS_PER_W)], idx_v)
    pltpu.sync_copy(bias_hbm, b_v)
    pltpu.async_copy(table_hbm.at[idx_v], rows_v, sem).wait()
    for r in range(_ROWS_PER_W):
        for c in range(D_SINGLE // 16):
            sl = pl.ds(c * 16, 16)
            rows_v[r, sl] = rows_v[r, sl] + b_v[sl]
    pltpu.sync_copy(rows_v, out_hbm.at[pl.ds(base, _ROWS_PER_W)])


def kernel(target_seq, W_dense, b_dense, relpos_table):
    single = pl.kernel(
        _single_sc_kernel,
        mesh=plsc.VectorSubcoreMesh(core_axis_name="c", subcore_axis_name="s"),
        out_type=jax.ShapeDtypeStruct((N_RES, D_SINGLE), jnp.float32),
        scratch_types=[
            pltpu.VMEM((_ROWS_PER_W,), jnp.int32),
            pltpu.VMEM((_ROWS_PER_W, D_SINGLE), jnp.float32),
            pltpu.VMEM((D_SINGLE,), jnp.float32),
            pltpu.SemaphoreType.DMA,
        ],
    )(W_dense, target_seq.astype(jnp.int32), b_dense)

    pair = pl.pallas_call(
        _pair_kernel,
        grid=(N_RES // BI,),
        in_specs=[pl.BlockSpec((2 * MAX_REL + 1, D_PAIR), lambda b: (0, 0))],
        out_specs=pl.BlockSpec((BI, N_RES, D_PAIR), lambda b: (b, 0, 0)),
        out_shape=jax.ShapeDtypeStruct((N_RES, N_RES, D_PAIR), jnp.float32),
        scratch_shapes=[pltpu.VMEM((PAD_ROWS, D_PAIR), jnp.float32)],
    )(relpos_table)

    return (single, pair)


# speedup vs baseline: 1.1097x; 1.0057x over previous
import jax
import jax.numpy as jnp
from jax import lax
from jax.experimental import pallas as pl
from jax.experimental.pallas import tpu as pltpu
from jax.experimental.pallas import tpu_sc as plsc

D_SINGLE = 384
D_PAIR = 128
NUM_AA = 21
MAX_REL = 32
N_RES = 1024

BI = 8
PAD_ROWS = 2 * N_RES + 2 * MAX_REL


_N_SEM = 8


def _pair_kernel(table_ref, out_hbm, padded_ref, sems):
    hi = table_ref[2 * MAX_REL, :]
    lo = table_ref[0, :]
    padded_ref[pl.ds(0, N_RES), :] = jnp.broadcast_to(hi, (N_RES, D_PAIR))
    for r in range(2 * MAX_REL + 1):
        padded_ref[N_RES + r, :] = table_ref[2 * MAX_REL - r, :]
    tail = PAD_ROWS - N_RES - 2 * MAX_REL - 1
    padded_ref[pl.ds(N_RES + 2 * MAX_REL + 1, tail), :] = jnp.broadcast_to(
        lo, (tail, D_PAIR)
    )

    def _copy(i, slot):
        return pltpu.make_async_copy(
            padded_ref.at[pl.ds((N_RES + MAX_REL) - i, N_RES), :],
            out_hbm.at[i],
            sems.at[slot],
        )

    def _body(i, carry):
        @pl.when(i >= _N_SEM)
        def _drain():
            _copy(i - _N_SEM, lax.rem(i, _N_SEM)).wait()

        _copy(i, lax.rem(i, _N_SEM)).start()
        return carry

    lax.fori_loop(0, N_RES, _body, 0)
    for s in range(_N_SEM):
        _copy(N_RES - _N_SEM + s, (N_RES + s) % _N_SEM).wait()


_N_SC_WORKERS = 32
_ROWS_PER_W = N_RES // _N_SC_WORKERS


def _single_sc_kernel(table_hbm, idx_hbm, bias_hbm, out_hbm, idx_v, rows_v, b_v, sem):
    wid = lax.axis_index("s") * 2 + lax.axis_index("c")
    base = wid * _ROWS_PER_W
    pltpu.sync_copy(idx_hbm.at[pl.ds(base, _ROWS_PER_W)], idx_v)
    pltpu.sync_copy(bias_hbm, b_v)
    pltpu.async_copy(table_hbm.at[idx_v], rows_v, sem).wait()
    for r in range(_ROWS_PER_W):
        for c in range(D_SINGLE // 16):
            sl = pl.ds(c * 16, 16)
            rows_v[r, sl] = rows_v[r, sl] + b_v[sl]
    pltpu.sync_copy(rows_v, out_hbm.at[pl.ds(base, _ROWS_PER_W)])


def kernel(target_seq, W_dense, b_dense, relpos_table):
    single = pl.kernel(
        _single_sc_kernel,
        mesh=plsc.VectorSubcoreMesh(core_axis_name="c", subcore_axis_name="s"),
        out_type=jax.ShapeDtypeStruct((N_RES, D_SINGLE), jnp.float32),
        scratch_types=[
            pltpu.VMEM((_ROWS_PER_W,), jnp.int32),
            pltpu.VMEM((_ROWS_PER_W, D_SINGLE), jnp.float32),
            pltpu.VMEM((D_SINGLE,), jnp.float32),
            pltpu.SemaphoreType.DMA,
        ],
    )(W_dense, target_seq.astype(jnp.int32), b_dense)

    pair = pl.pallas_call(
        _pair_kernel,
        in_specs=[pl.BlockSpec((2 * MAX_REL + 1, D_PAIR), lambda: (0, 0))],
        out_specs=pl.BlockSpec(memory_space=pl.ANY),
        out_shape=jax.ShapeDtypeStruct((N_RES, N_RES, D_PAIR), jnp.float32),
        scratch_shapes=[
            pltpu.VMEM((PAD_ROWS, D_PAIR), jnp.float32),
            pltpu.SemaphoreType.DMA((_N_SEM,)),
        ],
    )(relpos_table)

    return (single, pair)


# speedup vs baseline: 1.1170x; 1.0066x over previous
import jax
import jax.numpy as jnp
from jax import lax
from jax.experimental import pallas as pl
from jax.experimental.pallas import tpu as pltpu
from jax.experimental.pallas import tpu_sc as plsc

D_SINGLE = 384
D_PAIR = 128
NUM_AA = 21
MAX_REL = 32
N_RES = 1024

BI = 8
PAD_ROWS = 2 * N_RES + 2 * MAX_REL


_BB = 32
_N_BLK = N_RES // _BB
_BAND_W = 3 * MAX_REL
_CHUNK = 240
_N_SEM = 16
_N_BAND_BUF = 4


def _pair_kernel(table_ref, out_hbm, padded_ref, hi_buf, lo_buf, band_buf,
                 sems, band_sems):
    hi = table_ref[2 * MAX_REL, :]
    lo = table_ref[0, :]
    padded_ref[pl.ds(0, N_RES), :] = jnp.broadcast_to(hi, (N_RES, D_PAIR))
    for r in range(2 * MAX_REL + 1):
        padded_ref[N_RES + r, :] = table_ref[2 * MAX_REL - r, :]
    tail = PAD_ROWS - N_RES - 2 * MAX_REL - 1
    padded_ref[pl.ds(N_RES + 2 * MAX_REL + 1, tail), :] = jnp.broadcast_to(
        lo, (tail, D_PAIR)
    )
    hi_buf[...] = jnp.broadcast_to(hi, (_BB, _CHUNK, D_PAIR))
    lo_buf[...] = jnp.broadcast_to(lo, (_BB, _CHUNK, D_PAIR))

    pending = [None] * _N_SEM
    band_pending = [None] * _N_BAND_BUF
    n_dma = 0

    def _const_dma(buf, i0, j0, w):
        nonlocal n_dma
        slot = n_dma % _N_SEM
        n_dma += 1
        if pending[slot] is not None:
            pending[slot].wait()
        d = pltpu.make_async_copy(
            buf.at[:, pl.ds(0, w), :],
            out_hbm.at[pl.ds(i0, _BB), pl.ds(j0, w), :],
            sems.at[slot],
        )
        d.start()
        pending[slot] = d

    for k in range(_N_BLK):
        i0 = k * _BB
        jb = max(0, i0 - MAX_REL)
        je = min(N_RES, i0 + _BB + MAX_REL)
        wb = je - jb

        bslot = k % _N_BAND_BUF
        if band_pending[bslot] is not None:
            band_pending[bslot].wait()
        for r in range(_BB):
            src = (N_RES + MAX_REL) - (i0 + r) + jb
            band_buf[bslot, r, pl.ds(0, wb), :] = padded_ref[pl.ds(src, wb), :]
        d = pltpu.make_async_copy(
            band_buf.at[bslot, :, pl.ds(0, wb), :],
            out_hbm.at[pl.ds(i0, _BB), pl.ds(jb, wb), :],
            band_sems.at[bslot],
        )
        d.start()
        band_pending[bslot] = d

        off = 0
        while off < jb:
            w = min(_CHUNK, jb - off)
            _const_dma(hi_buf, i0, off, w)
            off += w
        off = je
        while off < N_RES:
            w = min(_CHUNK, N_RES - off)
            _const_dma(lo_buf, i0, off, w)
            off += w

    for d in pending:
        if d is not None:
            d.wait()
    for d in band_pending:
        if d is not None:
            d.wait()


_N_SC_WORKERS = 32
_ROWS_PER_W = N_RES // _N_SC_WORKERS


def _single_sc_kernel(table_hbm, idx_hbm, out_hbm, idx_v, rows_v, sem):
    wid = lax.axis_index("s") * 2 + lax.axis_index("c")
    base = wid * _ROWS_PER_W
    pltpu.sync_copy(idx_hbm.at[pl.ds(base, _ROWS_PER_W)], idx_v)
    pltpu.async_copy(table_hbm.at[idx_v], rows_v, sem).wait()
    pltpu.sync_copy(rows_v, out_hbm.at[pl.ds(base, _ROWS_PER_W)])


def kernel(target_seq, W_dense, b_dense, relpos_table):
    table_wb = W_dense + b_dense[None, :]
    single = pl.kernel(
        _single_sc_kernel,
        mesh=plsc.VectorSubcoreMesh(core_axis_name="c", subcore_axis_name="s"),
        out_type=jax.ShapeDtypeStruct((N_RES, D_SINGLE), jnp.float32),
        scratch_types=[
            pltpu.VMEM((_ROWS_PER_W,), jnp.int32),
            pltpu.VMEM((_ROWS_PER_W, D_SINGLE), jnp.float32),
            pltpu.SemaphoreType.DMA,
        ],
    )(table_wb, target_seq.astype(jnp.int32))

    pair = pl.pallas_call(
        _pair_kernel,
        in_specs=[pl.BlockSpec((2 * MAX_REL + 1, D_PAIR), lambda: (0, 0))],
        out_specs=pl.BlockSpec(memory_space=pl.ANY),
        out_shape=jax.ShapeDtypeStruct((N_RES, N_RES, D_PAIR), jnp.float32),
        scratch_shapes=[
            pltpu.VMEM((PAD_ROWS, D_PAIR), jnp.float32),
            pltpu.VMEM((_BB, _CHUNK, D_PAIR), jnp.float32),
            pltpu.VMEM((_BB, _CHUNK, D_PAIR), jnp.float32),
            pltpu.VMEM((_N_BAND_BUF, _BB, _BAND_W, D_PAIR), jnp.float32),
            pltpu.SemaphoreType.DMA((_N_SEM,)),
            pltpu.SemaphoreType.DMA((_N_BAND_BUF,)),
        ],
    )(relpos_table)

    return (single, pair)


# speedup vs baseline: 1.1999x; 1.0741x over previous
import jax
import jax.numpy as jnp
from jax import lax
from jax.experimental import pallas as pl
from jax.experimental.pallas import tpu as pltpu
from jax.experimental.pallas import tpu_sc as plsc

D_SINGLE = 384
D_PAIR = 128
NUM_AA = 21
MAX_REL = 32
N_RES = 1024

BI = 8
PAD_ROWS = 2 * N_RES + 2 * MAX_REL


_BB = 32
_N_BLK = N_RES // _BB
_BAND_W = 3 * MAX_REL
_CHUNK = 240
_N_SEM = 16
_N_BAND_BUF = 4


def _pair_single_kernel(table_ref, seq_ref, w_ref, b_ref, out_hbm, single_ref,
                        padded_ref, hi_buf, lo_buf, band_buf, sems, band_sems):
    def _single():
        seq = seq_ref[:, 0]
        oh = seq[:, None] == lax.broadcasted_iota(jnp.int32, (N_RES, NUM_AA), 1)
        single_ref[...] = (
            jnp.dot(oh.astype(jnp.float32), w_ref[...],
                    preferred_element_type=jnp.float32)
            + b_ref[0, :]
        )

    _pair_body(table_ref, out_hbm, padded_ref, hi_buf, lo_buf, band_buf,
               sems, band_sems, extra=_single)


def _pair_kernel(table_ref, out_hbm, padded_ref, hi_buf, lo_buf, band_buf,
                 sems, band_sems):
    _pair_body(table_ref, out_hbm, padded_ref, hi_buf, lo_buf, band_buf,
               sems, band_sems)


def _pair_body(table_ref, out_hbm, padded_ref, hi_buf, lo_buf, band_buf,
               sems, band_sems, extra=None):
    hi = table_ref[2 * MAX_REL, :]
    lo = table_ref[0, :]
    padded_ref[pl.ds(0, N_RES), :] = jnp.broadcast_to(hi, (N_RES, D_PAIR))
    for r in range(2 * MAX_REL + 1):
        padded_ref[N_RES + r, :] = table_ref[2 * MAX_REL - r, :]
    tail = PAD_ROWS - N_RES - 2 * MAX_REL - 1
    padded_ref[pl.ds(N_RES + 2 * MAX_REL + 1, tail), :] = jnp.broadcast_to(
        lo, (tail, D_PAIR)
    )
    hi_buf[...] = jnp.broadcast_to(hi, (_BB, _CHUNK, D_PAIR))
    lo_buf[...] = jnp.broadcast_to(lo, (_BB, _CHUNK, D_PAIR))

    pending = [None] * _N_SEM
    band_pending = [None] * _N_BAND_BUF
    n_dma = 0

    def _const_dma(buf, i0, j0, w):
        nonlocal n_dma
        slot = n_dma % _N_SEM
        n_dma += 1
        if pending[slot] is not None:
            pending[slot].wait()
        d = pltpu.make_async_copy(
            buf.at[:, pl.ds(0, w), :],
            out_hbm.at[pl.ds(i0, _BB), pl.ds(j0, w), :],
            sems.at[slot],
        )
        d.start()
        pending[slot] = d

    for k in range(_N_BLK):
        i0 = k * _BB
        jb = max(0, i0 - MAX_REL)
        je = min(N_RES, i0 + _BB + MAX_REL)
        wb = je - jb

        bslot = k % _N_BAND_BUF
        if band_pending[bslot] is not None:
            band_pending[bslot].wait()
        for r in range(_BB):
            src = (N_RES + MAX_REL) - (i0 + r) + jb
            band_buf[bslot, r, pl.ds(0, wb), :] = padded_ref[pl.ds(src, wb), :]
        d = pltpu.make_async_copy(
            band_buf.at[bslot, :, pl.ds(0, wb), :],
            out_hbm.at[pl.ds(i0, _BB), pl.ds(jb, wb), :],
            band_sems.at[bslot],
        )
        d.start()
        band_pending[bslot] = d

        off = 0
        while off < jb:
            w = min(_CHUNK, jb - off)
            _const_dma(hi_buf, i0, off, w)
            off += w
        off = je
        while off < N_RES:
            w = min(_CHUNK, N_RES - off)
            _const_dma(lo_buf, i0, off, w)
            off += w

    if extra is not None:
        extra()
    for d in pending:
        if d is not None:
            d.wait()
    for d in band_pending:
        if d is not None:
            d.wait()


_N_SC_WORKERS = 32
_ROWS_PER_W = N_RES // _N_SC_WORKERS


def _single_sc_kernel(table_hbm, idx_hbm, out_hbm, idx_v, rows_v, sem):
    wid = lax.axis_index("s") * 2 + lax.axis_index("c")
    base = wid * _ROWS_PER_W
    pltpu.sync_copy(idx_hbm.at[pl.ds(base, _ROWS_PER_W)], idx_v)
    pltpu.async_copy(table_hbm.at[idx_v], rows_v, sem).wait()
    pltpu.sync_copy(rows_v, out_hbm.at[pl.ds(base, _ROWS_PER_W)])


def kernel(target_seq, W_dense, b_dense, relpos_table):
    pair, single = _merged_tc(target_seq, W_dense, b_dense, relpos_table)
    return (single, pair)


def _merged_tc(target_seq, W_dense, b_dense, relpos_table):
    return pl.pallas_call(
        _pair_single_kernel,
        in_specs=[
            pl.BlockSpec((2 * MAX_REL + 1, D_PAIR), lambda: (0, 0)),
            pl.BlockSpec((N_RES, 1), lambda: (0, 0)),
            pl.BlockSpec((NUM_AA, D_SINGLE), lambda: (0, 0)),
            pl.BlockSpec((1, D_SINGLE), lambda: (0, 0)),
        ],
        out_specs=[
            pl.BlockSpec(memory_space=pl.ANY),
            pl.BlockSpec((N_RES, D_SINGLE), lambda: (0, 0)),
        ],
        out_shape=[
            jax.ShapeDtypeStruct((N_RES, N_RES, D_PAIR), jnp.float32),
            jax.ShapeDtypeStruct((N_RES, D_SINGLE), jnp.float32),
        ],
        scratch_shapes=[
            pltpu.VMEM((PAD_ROWS, D_PAIR), jnp.float32),
            pltpu.VMEM((_BB, _CHUNK, D_PAIR), jnp.float32),
            pltpu.VMEM((_BB, _CHUNK, D_PAIR), jnp.float32),
            pltpu.VMEM((_N_BAND_BUF, _BB, _BAND_W, D_PAIR), jnp.float32),
            pltpu.SemaphoreType.DMA((_N_SEM,)),
            pltpu.SemaphoreType.DMA((_N_BAND_BUF,)),
        ],
    )(
        relpos_table,
        target_seq.astype(jnp.int32).reshape(N_RES, 1),
        W_dense,
        b_dense.reshape(1, D_SINGLE),
    )
